# baseline (speedup 1.0000x reference)
import functools

import jax
import jax.numpy as jnp
from jax import lax
from jax.experimental import pallas as pl
from jax.experimental.pallas import tpu as pltpu
from jax.experimental.pallas import tpu_sc as plsc

B = 16384
V = 100000
D = 64
H = 128
NL = 5
NSEG = 3
CW = 16
KP = 120

NC, NS = 2, 16
NW = NC * NS
BW = B // NW
CG = 8
NCH = BW // CG

R1 = 2000


def _transform_body(chars_ref, wp_ref, wt_ref, out_ref, m_ref):
    @pl.when(pl.program_id(0) == 0)
    def _():
        wt = wt_ref[...]
        m_ent = jnp.dot(wp_ref[0:64, :], wt,
                        preferred_element_type=jnp.float32) * (1.0 / 20.0)
        m_left = jnp.dot(wp_ref[128:192, :], wt,
                         preferred_element_type=jnp.float32)
        m_right = jnp.dot(wp_ref[192:256, :], wt,
                          preferred_element_type=jnp.float32)
        m_ref[...] = jnp.concatenate([m_ent, m_left, m_right], axis=1)

    out_ref[...] = jnp.dot(chars_ref[...], m_ref[...],
                           preferred_element_type=jnp.float32
                           ).astype(jnp.bfloat16)


def _transform(chars_table, wp, wt_pad):
    return pl.pallas_call(
        _transform_body,
        grid=(V // R1,),
        in_specs=[
            pl.BlockSpec((R1, D), lambda i: (i, 0)),
            pl.BlockSpec((2 * H, H), lambda i: (0, 0)),
            pl.BlockSpec((H, CW), lambda i: (0, 0)),
        ],
        out_specs=pl.BlockSpec((R1, NSEG * CW), lambda i: (i, 0)),
        out_shape=jax.ShapeDtypeStruct((V, NSEG * CW), jnp.bfloat16),
        scratch_shapes=[pltpu.VMEM((D, NSEG * CW), jnp.float32)],
    )(chars_table, wp, wt_pad)


NRW = V * NSEG * CW // 128
NRP = 1250
NRC = 250


def _repack_body(tc1_hbm, tci_hbm, vin, vout):
    cid = lax.axis_index("c")
    sid = lax.axis_index("s")
    wid = sid * NC + cid

    @pl.when(wid < NRW // NRP)
    def _():
        def chunk(k, carry):
            r0 = wid * NRP + k * NRC
            pltpu.sync_copy(tc1_hbm.at[pl.ds(r0 * 128, NRC * 128)], vin)

            def row(i, c2):
                for t in range(4):
                    x = vin[pl.ds(i * 128 + 32 * t, 32)]
                    vout[pl.ds((i * 4 + t) * 2, 2), :] = x.reshape(2, CW)
                return c2

            lax.fori_loop(0, NRC, row, 0)
            pltpu.sync_copy(vout, tci_hbm.at[pl.ds(r0 * 8, NRC * 8)])
            return carry

        lax.fori_loop(0, NRP // NRC, chunk, 0)


@functools.partial(
    pl.kernel,
    mesh=plsc.VectorSubcoreMesh(core_axis_name="c", subcore_axis_name="s"),
    out_type=jax.ShapeDtypeStruct((NSEG * V, CW), jnp.bfloat16),
    scratch_types=[
        pltpu.VMEM((NRC * 128,), jnp.bfloat16),
        pltpu.VMEM((NRC * 8, CW), jnp.bfloat16),
    ],
    compiler_params=pltpu.CompilerParams(use_tc_tiling_on_sc=False),
)
def _sc_repack(tc1_hbm, tci_hbm, vin, vout):
    _repack_body(tc1_hbm, tci_hbm, vin, vout)


IRC = 64


def _idxstage_body(idx2_hbm, idxf_hbm, vin, vout):
    cid = lax.axis_index("c")
    sid = lax.axis_index("s")
    base = (sid * NC + cid) * BW

    def chunk(k, carry):
        b0 = base + k * IRC
        pltpu.sync_copy(idx2_hbm.at[pl.ds(b0, IRC), :], vin)

        def row(r, c2):
            for c in range(7):
                vout[pl.ds(r * KP + 16 * c, 16)] = vin[r, pl.ds(16 * c, 16)]
            vout[pl.ds(r * KP + 104, 16)] = vin[r, pl.ds(104, 16)]
            return c2

        lax.fori_loop(0, IRC, row, 0)
        pltpu.sync_copy(vout, idxf_hbm.at[pl.ds(b0 * KP, IRC * KP)])
        return carry

    lax.fori_loop(0, BW // IRC, chunk, 0)


@functools.partial(
    pl.kernel,
    mesh=plsc.VectorSubcoreMesh(core_axis_name="c", subcore_axis_name="s"),
    out_type=jax.ShapeDtypeStruct((B * KP,), jnp.int32),
    scratch_types=[
        pltpu.VMEM((IRC, KP), jnp.int32),
        pltpu.VMEM((IRC * KP,), jnp.int32),
    ],
    compiler_params=pltpu.CompilerParams(use_tc_tiling_on_sc=True),
)
def _sc_idxstage(idx2_hbm, idxf_hbm, vin, vout):
    _idxstage_body(idx2_hbm, idxf_hbm, vin, vout)


def _sc_body(idx_hbm, tc_hbm, out_hbm, idx_v, rows_v, acc_v, sem0, sem1):
    cid = lax.axis_index("c")
    sid = lax.axis_index("s")
    base = (sid * NC + cid) * BW

    pltpu.sync_copy(idx_hbm.at[pl.ds(base * KP, BW * KP)], idx_v)

    def copies(ci, buf, sem):
        return [
            pltpu.make_async_copy(
                tc_hbm.at[idx_v.at[pl.ds(ci * CG * KP, CG * KP)]],
                rows_v.at[buf], sem)
        ]

    def fire(ci, buf, sem):
        for c in copies(ci, buf, sem):
            c.start()

    def drain_reduce(ci, buf, sem):
        for c in copies(ci, buf, sem):
            c.wait()
        for e in range(CG):
            r = e * KP
            a0 = rows_v[buf, pl.ds(r + 0, 2), :]
            a1 = rows_v[buf, pl.ds(r + 2, 2), :]
            a2 = rows_v[buf, pl.ds(r + 4, 2), :]
            a3 = rows_v[buf, pl.ds(r + 6, 2), :]
            for j in range(8, KP, 8):
                a0 = a0 + rows_v[buf, pl.ds(r + j + 0, 2), :]
                a1 = a1 + rows_v[buf, pl.ds(r + j + 2, 2), :]
                a2 = a2 + rows_v[buf, pl.ds(r + j + 4, 2), :]
                a3 = a3 + rows_v[buf, pl.ds(r + j + 6, 2), :]
            acc_v[pl.ds(2 * e, 2), :] = (a0 + a1) + (a2 + a3)
        pltpu.sync_copy(acc_v,
                        out_hbm.at[pl.ds((base + ci * CG) * 2, CG * 2)])

    fire(0, 0, sem0)

    def pair(p, carry):
        ci = 2 * p
        fire(ci + 1, 1, sem1)
        drain_reduce(ci, 0, sem0)

        @pl.when(ci + 2 < NCH)
        def _():
            fire(ci + 2, 0, sem0)

        drain_reduce(ci + 1, 1, sem1)
        return carry

    lax.fori_loop(0, NCH // 2, pair, 0)


@functools.partial(
    pl.kernel,
    mesh=plsc.VectorSubcoreMesh(core_axis_name="c", subcore_axis_name="s"),
    out_type=jax.ShapeDtypeStruct((2 * B, CW), jnp.bfloat16),
    scratch_types=[
        pltpu.VMEM((BW * KP,), jnp.int32),
        pltpu.VMEM((2, CG * KP, CW), jnp.bfloat16),
        pltpu.VMEM((2 * CG, CW), jnp.bfloat16),
        pltpu.SemaphoreType.DMA,
        pltpu.SemaphoreType.DMA,
    ],
    compiler_params=pltpu.CompilerParams(use_tc_tiling_on_sc=False),
)
def _sc_gather_pool(idx_hbm, tc_hbm, out_hbm, idx_v, rows_v, acc_v, sem0,
                    sem1):
    _sc_body(idx_hbm, tc_hbm, out_hbm, idx_v, rows_v, acc_v, sem0, sem1)


BB = 2048


def _loss_body(logits_ref, lab_ref, tgt_ref, ltab_ref, wp_ref, wt_ref,
               bt_ref, bp_ref, out_ref):
    i = pl.program_id(0)
    lc = logits_ref[...].astype(jnp.float32).reshape(BB, 2, CW)
    logits_c = lc[:, 0, :] + lc[:, 1, :]
    wt = wt_ref[...]
    m_lab = jnp.dot(wp_ref[64:128, :], wt,
                    preferred_element_type=jnp.float32)
    tl = jnp.dot(ltab_ref[...], m_lab,
                 preferred_element_type=jnp.float32)

    iota_v = lax.broadcasted_iota(jnp.int32, (BB, 64), 1)
    cnt = jnp.zeros((BB, 64), jnp.float32)
    for j in range(4):
        cnt = cnt + (lab_ref[:, j:j + 1] == iota_v).astype(jnp.float32)
    lab_part = jnp.dot(cnt, tl, preferred_element_type=jnp.float32)

    cvec = jnp.dot(bp_ref[...], wt,
                   preferred_element_type=jnp.float32) + bt_ref[...]
    logits = logits_c + lab_part + cvec

    e = jnp.exp(logits)
    ii = lax.broadcasted_iota(jnp.int32, (CW, CW), 0)
    jj = lax.broadcasted_iota(jnp.int32, (CW, CW), 1)
    pmat = ((ii // 2 == jj // 2) & (jj < 2 * NL)).astype(jnp.float32)
    psum = jnp.dot(e, pmat, preferred_element_type=jnp.float32)
    lane = lax.broadcasted_iota(jnp.int32, (BB, CW), 1)
    even10 = ((lane % 2 == 0) & (lane < 2 * NL)).astype(jnp.float32)
    lse_sum = jnp.sum(jnp.log(jnp.where(psum > 0.0, psum, 1.0)) * even10)

    emat = ((jj // 2 == ii) & (jj < 2 * NL)).astype(jnp.float32)
    tlane = jnp.dot(tgt_ref[...], emat, preferred_element_type=jnp.float32)
    isodd = (lane % 2).astype(jnp.float32)
    in10 = (lane < 2 * NL).astype(jnp.float32)
    sel = (tlane * isodd + (1.0 - tlane) * (1.0 - isodd)) * in10
    sel_sum = jnp.sum(sel * logits)

    partial = (lse_sum - sel_sum) * (1.0 / (B * NL))

    @pl.when(i == 0)
    def _():
        out_ref[...] = jnp.zeros((1, 1), jnp.float32)

    out_ref[...] = out_ref[...] + jnp.full((1, 1), 1.0) * partial


def _loss(logits_c, entity_label, tgt_pad, label_table, wp, wt_pad, bt_pad,
          bp2):
    return pl.pallas_call(
        _loss_body,
        grid=(B // BB,),
        in_specs=[
            pl.BlockSpec((2 * BB, CW), lambda i: (i, 0)),
            pl.BlockSpec((BB, 4), lambda i: (i, 0)),
            pl.BlockSpec((BB, CW), lambda i: (i, 0)),
            pl.BlockSpec((64, 64), lambda i: (0, 0)),
            pl.BlockSpec((2 * H, H), lambda i: (0, 0)),
            pl.BlockSpec((H, CW), lambda i: (0, 0)),
            pl.BlockSpec((1, CW), lambda i: (0, 0)),
            pl.BlockSpec((1, H), lambda i: (0, 0)),
        ],
        out_specs=pl.BlockSpec((1, 1), lambda i: (0, 0)),
        out_shape=jax.ShapeDtypeStruct((1, 1), jnp.float32),
    )(logits_c, entity_label, tgt_pad, label_table, wp, wt_pad, bt_pad, bp2)


def kernel(left_chars, right_chars, entity_chars, entity_label, target,
           chars_table, label_table, Wp, bp, Wt, bt):
    wt_flat = Wt.transpose(1, 0, 2).reshape(H, 2 * NL)
    wt_pad = jnp.zeros((H, CW), jnp.float32).at[:, :2 * NL].set(wt_flat)
    bt_pad = jnp.zeros((1, CW), jnp.float32).at[0, :2 * NL].set(
        bt.reshape(-1))
    tgt_pad = jnp.zeros((B, CW), jnp.float32).at[:, :NL].set(
        target.astype(jnp.float32))

    t48 = _transform(chars_table, Wp, wt_pad)
    tc1 = t48.reshape(-1)
    tc = _sc_repack(tc1)

    idx2 = jnp.concatenate(
        [entity_chars * 3, left_chars * 3 + 1, right_chars * 3 + 2],
        axis=1)
    idxc = _sc_idxstage(idx2)

    logits_c = _sc_gather_pool(idxc, tc)

    out = _loss(logits_c, entity_label, tgt_pad, label_table, Wp, wt_pad,
                bt_pad, bp.reshape(1, H))
    return out[0, 0]


# speedup vs baseline: 1.0769x; 1.0769x over previous
import functools

import jax
import jax.numpy as jnp
from jax import lax
from jax.experimental import pallas as pl
from jax.experimental.pallas import tpu as pltpu
from jax.experimental.pallas import tpu_sc as plsc

B = 16384
V = 100000
D = 64
H = 128
NL = 5
NSEG = 3
CW = 16
KP = 120

NC, NS = 2, 16
NW = NC * NS
BW = B // NW
CG = 8
NCH = BW // CG

R1 = 2000


def _transform_body(chars_ref, wp_ref, wt_ref, out_ref, m_ref):
    @pl.when(pl.program_id(0) == 0)
    def _():
        wt = wt_ref[...]
        m_ent = jnp.dot(wp_ref[0:64, :], wt,
                        preferred_element_type=jnp.float32) * (1.0 / 20.0)
        m_left = jnp.dot(wp_ref[128:192, :], wt,
                         preferred_element_type=jnp.float32)
        m_right = jnp.dot(wp_ref[192:256, :], wt,
                          preferred_element_type=jnp.float32)
        m_ref[...] = jnp.concatenate([m_ent, m_left, m_right], axis=1)

    out_ref[...] = jnp.dot(chars_ref[...], m_ref[...],
                           preferred_element_type=jnp.float32
                           ).astype(jnp.bfloat16)


def _transform(chars_table, wp, wt_pad):
    return pl.pallas_call(
        _transform_body,
        grid=(V // R1,),
        in_specs=[
            pl.BlockSpec((R1, D), lambda i: (i, 0)),
            pl.BlockSpec((2 * H, H), lambda i: (0, 0)),
            pl.BlockSpec((H, CW), lambda i: (0, 0)),
        ],
        out_specs=pl.BlockSpec((R1, NSEG * CW), lambda i: (i, 0)),
        out_shape=jax.ShapeDtypeStruct((V, NSEG * CW), jnp.bfloat16),
        scratch_shapes=[pltpu.VMEM((D, NSEG * CW), jnp.float32)],
    )(chars_table, wp, wt_pad)


NRW = V * NSEG * CW // 128
NRP = 1250
NRC = 250


def _repack_body(tc1_hbm, tci_hbm, vin, vout):
    cid = lax.axis_index("c")
    sid = lax.axis_index("s")
    wid = sid * NC + cid

    @pl.when(wid < NRW // NRP)
    def _():
        def chunk(k, carry):
            r0 = wid * NRP + k * NRC
            pltpu.sync_copy(tc1_hbm.at[pl.ds(r0 * 128, NRC * 128)], vin)

            def row(i, c2):
                for t in range(4):
                    x = vin[pl.ds(i * 128 + 32 * t, 32)]
                    vout[pl.ds((i * 4 + t) * 2, 2), :] = x.reshape(2, CW)
                return c2

            lax.fori_loop(0, NRC, row, 0)
            pltpu.sync_copy(vout, tci_hbm.at[pl.ds(r0 * 8, NRC * 8)])
            return carry

        lax.fori_loop(0, NRP // NRC, chunk, 0)


@functools.partial(
    pl.kernel,
    mesh=plsc.VectorSubcoreMesh(core_axis_name="c", subcore_axis_name="s"),
    out_type=jax.ShapeDtypeStruct((NSEG * V, CW), jnp.bfloat16),
    scratch_types=[
        pltpu.VMEM((NRC * 128,), jnp.bfloat16),
        pltpu.VMEM((NRC * 8, CW), jnp.bfloat16),
    ],
    compiler_params=pltpu.CompilerParams(use_tc_tiling_on_sc=False),
)
def _sc_repack(tc1_hbm, tci_hbm, vin, vout):
    _repack_body(tc1_hbm, tci_hbm, vin, vout)


IRC = 64


def _idxstage_body(idx2_hbm, idxf_hbm, vin, vout):
    cid = lax.axis_index("c")
    sid = lax.axis_index("s")
    base = (sid * NC + cid) * BW

    def chunk(k, carry):
        b0 = base + k * IRC
        pltpu.sync_copy(idx2_hbm.at[pl.ds(b0, IRC), :], vin)

        def row(r, c2):
            for c in range(7):
                vout[pl.ds(r * KP + 16 * c, 16)] = vin[r, pl.ds(16 * c, 16)]
            vout[pl.ds(r * KP + 104, 16)] = vin[r, pl.ds(104, 16)]
            return c2

        lax.fori_loop(0, IRC, row, 0)
        pltpu.sync_copy(vout, idxf_hbm.at[pl.ds(b0 * KP, IRC * KP)])
        return carry

    lax.fori_loop(0, BW // IRC, chunk, 0)


@functools.partial(
    pl.kernel,
    mesh=plsc.VectorSubcoreMesh(core_axis_name="c", subcore_axis_name="s"),
    out_type=jax.ShapeDtypeStruct((B * KP,), jnp.int32),
    scratch_types=[
        pltpu.VMEM((IRC, KP), jnp.int32),
        pltpu.VMEM((IRC * KP,), jnp.int32),
    ],
    compiler_params=pltpu.CompilerParams(use_tc_tiling_on_sc=True),
)
def _sc_idxstage(idx2_hbm, idxf_hbm, vin, vout):
    _idxstage_body(idx2_hbm, idxf_hbm, vin, vout)


def _sc_body(idx_hbm, tc_hbm, out_hbm, idx_v, rows_v, acc_v, sem0, sem1):
    cid = lax.axis_index("c")
    sid = lax.axis_index("s")
    base = (sid * NC + cid) * BW

    pltpu.sync_copy(idx_hbm.at[pl.ds(base * KP, BW * KP)], idx_v)

    def copies(ci, buf, sem):
        return [
            pltpu.make_async_copy(
                tc_hbm.at[idx_v.at[pl.ds(ci * CG * KP, CG * KP)]],
                rows_v.at[buf], sem)
        ]

    def fire(ci, buf, sem):
        for c in copies(ci, buf, sem):
            c.start()

    def drain_reduce(ci, buf, sem):
        for c in copies(ci, buf, sem):
            c.wait()
        for e in range(CG):
            r = e * KP
            a0 = rows_v[buf, pl.ds(r + 0, 2), :]
            a1 = rows_v[buf, pl.ds(r + 2, 2), :]
            a2 = rows_v[buf, pl.ds(r + 4, 2), :]
            a3 = rows_v[buf, pl.ds(r + 6, 2), :]
            for j in range(8, KP, 8):
                a0 = a0 + rows_v[buf, pl.ds(r + j + 0, 2), :]
                a1 = a1 + rows_v[buf, pl.ds(r + j + 2, 2), :]
                a2 = a2 + rows_v[buf, pl.ds(r + j + 4, 2), :]
                a3 = a3 + rows_v[buf, pl.ds(r + j + 6, 2), :]
            acc_v[pl.ds(2 * e, 2), :] = (a0 + a1) + (a2 + a3)
        pltpu.sync_copy(acc_v,
                        out_hbm.at[pl.ds((base + ci * CG) * 2, CG * 2)])

    fire(0, 0, sem0)

    def pair(p, carry):
        ci = 2 * p
        fire(ci + 1, 1, sem1)
        drain_reduce(ci, 0, sem0)

        @pl.when(ci + 2 < NCH)
        def _():
            fire(ci + 2, 0, sem0)

        drain_reduce(ci + 1, 1, sem1)
        return carry

    lax.fori_loop(0, NCH // 2, pair, 0)


@functools.partial(
    pl.kernel,
    mesh=plsc.VectorSubcoreMesh(core_axis_name="c", subcore_axis_name="s"),
    out_type=jax.ShapeDtypeStruct((2 * B, CW), jnp.bfloat16),
    scratch_types=[
        pltpu.VMEM((BW * KP,), jnp.int32),
        pltpu.VMEM((2, CG * KP, CW), jnp.bfloat16),
        pltpu.VMEM((2 * CG, CW), jnp.bfloat16),
        pltpu.SemaphoreType.DMA,
        pltpu.SemaphoreType.DMA,
    ],
    compiler_params=pltpu.CompilerParams(use_tc_tiling_on_sc=False),
)
def _sc_gather_pool(idx_hbm, tc_hbm, out_hbm, idx_v, rows_v, acc_v, sem0,
                    sem1):
    _sc_body(idx_hbm, tc_hbm, out_hbm, idx_v, rows_v, acc_v, sem0, sem1)


BB = 2048


def _loss_body(logits_ref, lab_ref, tgt_ref, ltab_ref, wp_ref, wt_ref,
               bt_ref, bp_ref, out_ref):
    i = pl.program_id(0)
    lc = logits_ref[...].astype(jnp.float32).reshape(BB, 2, CW)
    logits_c = lc[:, 0, :] + lc[:, 1, :]
    wt = wt_ref[...]
    m_lab = jnp.dot(wp_ref[64:128, :], wt,
                    preferred_element_type=jnp.float32)
    tl = jnp.dot(ltab_ref[...], m_lab,
                 preferred_element_type=jnp.float32)

    iota_v = lax.broadcasted_iota(jnp.int32, (BB, 64), 1)
    cnt = jnp.zeros((BB, 64), jnp.float32)
    for j in range(4):
        cnt = cnt + (lab_ref[:, j:j + 1] == iota_v).astype(jnp.float32)
    lab_part = jnp.dot(cnt, tl, preferred_element_type=jnp.float32)

    cvec = jnp.dot(bp_ref[...], wt,
                   preferred_element_type=jnp.float32) + bt_ref[...]
    logits = logits_c + lab_part + cvec

    e = jnp.exp(logits)
    ii = lax.broadcasted_iota(jnp.int32, (CW, CW), 0)
    jj = lax.broadcasted_iota(jnp.int32, (CW, CW), 1)
    pmat = ((ii // 2 == jj // 2) & (jj < 2 * NL)).astype(jnp.float32)
    psum = jnp.dot(e, pmat, preferred_element_type=jnp.float32)
    lane = lax.broadcasted_iota(jnp.int32, (BB, CW), 1)
    even10 = ((lane % 2 == 0) & (lane < 2 * NL)).astype(jnp.float32)
    lse_sum = jnp.sum(jnp.log(jnp.where(psum > 0.0, psum, 1.0)) * even10)

    emat = ((jj // 2 == ii) & (jj < 2 * NL)).astype(jnp.float32)
    tlane = jnp.dot(tgt_ref[...], emat, preferred_element_type=jnp.float32)
    isodd = (lane % 2).astype(jnp.float32)
    in10 = (lane < 2 * NL).astype(jnp.float32)
    sel = (tlane * isodd + (1.0 - tlane) * (1.0 - isodd)) * in10
    sel_sum = jnp.sum(sel * logits)

    partial = (lse_sum - sel_sum) * (1.0 / (B * NL))

    @pl.when(i == 0)
    def _():
        out_ref[...] = jnp.zeros((1, 1), jnp.float32)

    out_ref[...] = out_ref[...] + jnp.full((1, 1), 1.0) * partial


def _loss(logits_c, entity_label, tgt_pad, label_table, wp, wt_pad, bt_pad,
          bp2):
    return pl.pallas_call(
        _loss_body,
        grid=(B // BB,),
        in_specs=[
            pl.BlockSpec((2 * BB, CW), lambda i: (i, 0)),
            pl.BlockSpec((BB, 4), lambda i: (i, 0)),
            pl.BlockSpec((BB, CW), lambda i: (i, 0)),
            pl.BlockSpec((64, 64), lambda i: (0, 0)),
            pl.BlockSpec((2 * H, H), lambda i: (0, 0)),
            pl.BlockSpec((H, CW), lambda i: (0, 0)),
            pl.BlockSpec((1, CW), lambda i: (0, 0)),
            pl.BlockSpec((1, H), lambda i: (0, 0)),
        ],
        out_specs=pl.BlockSpec((1, 1), lambda i: (0, 0)),
        out_shape=jax.ShapeDtypeStruct((1, 1), jnp.float32),
    )(logits_c, entity_label, tgt_pad, label_table, wp, wt_pad, bt_pad, bp2)


def kernel(left_chars, right_chars, entity_chars, entity_label, target,
           chars_table, label_table, Wp, bp, Wt, bt):
    wt_flat = Wt.transpose(1, 0, 2).reshape(H, 2 * NL)
    wt_pad = jnp.zeros((H, CW), jnp.float32).at[:, :2 * NL].set(wt_flat)
    bt_pad = jnp.zeros((1, CW), jnp.float32).at[0, :2 * NL].set(
        bt.reshape(-1))
    tgt_pad = jnp.zeros((B, CW), jnp.float32).at[:, :NL].set(
        target.astype(jnp.float32))

    t48 = _transform(chars_table, Wp, wt_pad)
    tc = t48.reshape(V * NSEG, CW)

    idx2 = jnp.concatenate(
        [entity_chars * 3, left_chars * 3 + 1, right_chars * 3 + 2],
        axis=1)
    idxc = _sc_idxstage(idx2)

    logits_c = _sc_gather_pool(idxc, tc)

    out = _loss(logits_c, entity_label, tgt_pad, label_table, Wp, wt_pad,
                bt_pad, bp.reshape(1, H))
    return out[0, 0]


# speedup vs baseline: 1.1344x; 1.0534x over previous
import functools

import jax
import jax.numpy as jnp
from jax import lax
from jax.experimental import pallas as pl
from jax.experimental.pallas import tpu as pltpu
from jax.experimental.pallas import tpu_sc as plsc

B = 16384
V = 100000
D = 64
H = 128
NL = 5
NSEG = 3
CW = 16
KP = 120

NC, NS = 2, 16
NW = NC * NS
BW = B // NW
CG = 8
NCH = BW // CG

R1 = 2000


def _transform_body(chars_ref, wp_ref, wt_ref, out_ref, m_ref):
    @pl.when(pl.program_id(0) == 0)
    def _():
        wt = wt_ref[...]
        m_ent = jnp.dot(wp_ref[0:64, :], wt,
                        preferred_element_type=jnp.float32) * (1.0 / 20.0)
        m_left = jnp.dot(wp_ref[128:192, :], wt,
                         preferred_element_type=jnp.float32)
        m_right = jnp.dot(wp_ref[192:256, :], wt,
                          preferred_element_type=jnp.float32)
        m_ref[...] = jnp.concatenate([m_ent, m_left, m_right], axis=1)

    out_ref[...] = jnp.dot(chars_ref[...], m_ref[...],
                           preferred_element_type=jnp.float32
                           ).astype(jnp.bfloat16)


def _transform(chars_table, wp, wt_pad):
    return pl.pallas_call(
        _transform_body,
        grid=(V // R1,),
        in_specs=[
            pl.BlockSpec((R1, D), lambda i: (i, 0)),
            pl.BlockSpec((2 * H, H), lambda i: (0, 0)),
            pl.BlockSpec((H, CW), lambda i: (0, 0)),
        ],
        out_specs=pl.BlockSpec((R1, NSEG * CW), lambda i: (i, 0)),
        out_shape=jax.ShapeDtypeStruct((V, NSEG * CW), jnp.bfloat16),
        scratch_shapes=[pltpu.VMEM((D, NSEG * CW), jnp.float32)],
    )(chars_table, wp, wt_pad)


def _sc_body(idx_hbm, tc_hbm, out_hbm, idx_v, rows_v, acc_v, sem0, sem1):
    cid = lax.axis_index("c")
    sid = lax.axis_index("s")
    base = (sid * NC + cid) * BW

    pltpu.sync_copy(idx_hbm.at[pl.ds(base * KP, BW * KP)], idx_v)

    def copies(ci, buf, sem):
        return [
            pltpu.make_async_copy(
                tc_hbm.at[idx_v.at[pl.ds(ci * CG * KP, CG * KP)]],
                rows_v.at[buf], sem)
        ]

    def fire(ci, buf, sem):
        for c in copies(ci, buf, sem):
            c.start()

    def drain_reduce(ci, buf, sem):
        for c in copies(ci, buf, sem):
            c.wait()
        for e in range(CG):
            r = e * KP
            a0 = rows_v[buf, pl.ds(r + 0, 2), :]
            a1 = rows_v[buf, pl.ds(r + 2, 2), :]
            a2 = rows_v[buf, pl.ds(r + 4, 2), :]
            a3 = rows_v[buf, pl.ds(r + 6, 2), :]
            for j in range(8, KP, 8):
                a0 = a0 + rows_v[buf, pl.ds(r + j + 0, 2), :]
                a1 = a1 + rows_v[buf, pl.ds(r + j + 2, 2), :]
                a2 = a2 + rows_v[buf, pl.ds(r + j + 4, 2), :]
                a3 = a3 + rows_v[buf, pl.ds(r + j + 6, 2), :]
            acc_v[pl.ds(2 * e, 2), :] = (a0 + a1) + (a2 + a3)
        pltpu.sync_copy(acc_v,
                        out_hbm.at[pl.ds((base + ci * CG) * 2, CG * 2)])

    fire(0, 0, sem0)

    def pair(p, carry):
        ci = 2 * p
        fire(ci + 1, 1, sem1)
        drain_reduce(ci, 0, sem0)

        @pl.when(ci + 2 < NCH)
        def _():
            fire(ci + 2, 0, sem0)

        drain_reduce(ci + 1, 1, sem1)
        return carry

    lax.fori_loop(0, NCH // 2, pair, 0)


@functools.partial(
    pl.kernel,
    mesh=plsc.VectorSubcoreMesh(core_axis_name="c", subcore_axis_name="s"),
    out_type=jax.ShapeDtypeStruct((2 * B, CW), jnp.bfloat16),
    scratch_types=[
        pltpu.VMEM((BW * KP,), jnp.int32),
        pltpu.VMEM((2, CG * KP, CW), jnp.bfloat16),
        pltpu.VMEM((2 * CG, CW), jnp.bfloat16),
        pltpu.SemaphoreType.DMA,
        pltpu.SemaphoreType.DMA,
    ],
    compiler_params=pltpu.CompilerParams(use_tc_tiling_on_sc=False),
)
def _sc_gather_pool(idx_hbm, tc_hbm, out_hbm, idx_v, rows_v, acc_v, sem0,
                    sem1):
    _sc_body(idx_hbm, tc_hbm, out_hbm, idx_v, rows_v, acc_v, sem0, sem1)


BB = 2048


def _loss_body(logits_ref, lab_ref, tgt_ref, ltab_ref, wp_ref, wt_ref,
               bt_ref, bp_ref, out_ref):
    i = pl.program_id(0)
    lc = logits_ref[...].astype(jnp.float32).reshape(BB, 2, CW)
    logits_c = lc[:, 0, :] + lc[:, 1, :]
    wt = wt_ref[...]
    m_lab = jnp.dot(wp_ref[64:128, :], wt,
                    preferred_element_type=jnp.float32)
    tl = jnp.dot(ltab_ref[...], m_lab,
                 preferred_element_type=jnp.float32)

    iota_v = lax.broadcasted_iota(jnp.int32, (BB, 64), 1)
    cnt = jnp.zeros((BB, 64), jnp.float32)
    for j in range(4):
        cnt = cnt + (lab_ref[:, j:j + 1] == iota_v).astype(jnp.float32)
    lab_part = jnp.dot(cnt, tl, preferred_element_type=jnp.float32)

    cvec = jnp.dot(bp_ref[...], wt,
                   preferred_element_type=jnp.float32) + bt_ref[...]
    logits = logits_c + lab_part + cvec

    e = jnp.exp(logits)
    ii = lax.broadcasted_iota(jnp.int32, (CW, CW), 0)
    jj = lax.broadcasted_iota(jnp.int32, (CW, CW), 1)
    pmat = ((ii // 2 == jj // 2) & (jj < 2 * NL)).astype(jnp.float32)
    psum = jnp.dot(e, pmat, preferred_element_type=jnp.float32)
    lane = lax.broadcasted_iota(jnp.int32, (BB, CW), 1)
    even10 = ((lane % 2 == 0) & (lane < 2 * NL)).astype(jnp.float32)
    lse_sum = jnp.sum(jnp.log(jnp.where(psum > 0.0, psum, 1.0)) * even10)

    emat = ((jj // 2 == ii) & (jj < 2 * NL)).astype(jnp.float32)
    tlane = jnp.dot(tgt_ref[...], emat, preferred_element_type=jnp.float32)
    isodd = (lane % 2).astype(jnp.float32)
    in10 = (lane < 2 * NL).astype(jnp.float32)
    sel = (tlane * isodd + (1.0 - tlane) * (1.0 - isodd)) * in10
    sel_sum = jnp.sum(sel * logits)

    partial = (lse_sum - sel_sum) * (1.0 / (B * NL))

    @pl.when(i == 0)
    def _():
        out_ref[...] = jnp.zeros((1, 1), jnp.float32)

    out_ref[...] = out_ref[...] + jnp.full((1, 1), 1.0) * partial


def _loss(logits_c, entity_label, tgt_pad, label_table, wp, wt_pad, bt_pad,
          bp2):
    return pl.pallas_call(
        _loss_body,
        grid=(B // BB,),
        in_specs=[
            pl.BlockSpec((2 * BB, CW), lambda i: (i, 0)),
            pl.BlockSpec((BB, 4), lambda i: (i, 0)),
            pl.BlockSpec((BB, CW), lambda i: (i, 0)),
            pl.BlockSpec((64, 64), lambda i: (0, 0)),
            pl.BlockSpec((2 * H, H), lambda i: (0, 0)),
            pl.BlockSpec((H, CW), lambda i: (0, 0)),
            pl.BlockSpec((1, CW), lambda i: (0, 0)),
            pl.BlockSpec((1, H), lambda i: (0, 0)),
        ],
        out_specs=pl.BlockSpec((1, 1), lambda i: (0, 0)),
        out_shape=jax.ShapeDtypeStruct((1, 1), jnp.float32),
    )(logits_c, entity_label, tgt_pad, label_table, wp, wt_pad, bt_pad, bp2)


def kernel(left_chars, right_chars, entity_chars, entity_label, target,
           chars_table, label_table, Wp, bp, Wt, bt):
    wt_flat = Wt.transpose(1, 0, 2).reshape(H, 2 * NL)
    wt_pad = jnp.zeros((H, CW), jnp.float32).at[:, :2 * NL].set(wt_flat)
    bt_pad = jnp.zeros((1, CW), jnp.float32).at[0, :2 * NL].set(
        bt.reshape(-1))
    tgt_pad = jnp.zeros((B, CW), jnp.float32).at[:, :NL].set(
        target.astype(jnp.float32))

    t48 = _transform(chars_table, Wp, wt_pad)
    tc = t48.reshape(V * NSEG, CW)

    idxc = jnp.concatenate(
        [entity_chars * 3, left_chars * 3 + 1, right_chars * 3 + 2],
        axis=1).reshape(-1)

    logits_c = _sc_gather_pool(idxc, tc)

    out = _loss(logits_c, entity_label, tgt_pad, label_table, Wp, wt_pad,
                bt_pad, bp.reshape(1, H))
    return out[0, 0]
